# Initial kernel scaffold; baseline (speedup 1.0000x reference)
import jax
import jax.numpy as jnp
from jax.experimental import pallas as pl


def kernel(x, edge_index, edge_weight, W1_rel, b1_rel, W1_root, bn_gamma, bn_beta, bn_mean, bn_var, W2_rel, b2_rel, W2_root, Wl, bl):
    raise NotImplementedError("write your pallas kernel here")


# speedup vs baseline: 21.7741x; 21.7741x over previous
import functools

import jax
import jax.numpy as jnp
from jax import lax
from jax.experimental import pallas as pl
from jax.experimental.pallas import tpu as pltpu
from jax.experimental.pallas import tpu_sc as plsc

N = 10000
E = 320000
H1 = 256
H2 = 16
BN_EPS = 1e-5

NC = 2
NS = 16
TILES = NC * NS
EDGES_PER_TILE = E // TILES
C = 80
K = EDGES_PER_TILE // C


def _sc1_body(x_hbm, src_hbm, dst_hbm, ew_hbm, zeros_hbm, out_hbm,
              x_v, src_v, dst_v, ew_v, msg_v, acc_sh, sem):
    c = lax.axis_index("c")
    s = lax.axis_index("s")
    row = c * NS + s

    pltpu.sync_copy(x_hbm, x_v)
    pltpu.sync_copy(src_hbm.at[row], src_v)
    pltpu.sync_copy(dst_hbm.at[row], dst_v)
    pltpu.sync_copy(ew_hbm.at[row], ew_v)

    @pl.when(s == 0)
    def _():
        pltpu.sync_copy(zeros_hbm, acc_sh)

    plsc.subcore_barrier()

    def chunk(j, carry):
        for g in range(C // 16):
            idx = src_v[j, pl.ds(g * 16, 16)]
            ew16 = ew_v[j, pl.ds(g * 16, 16)]
            vals = plsc.load_gather(x_v, [idx])
            msg_v[j, pl.ds(g * 16, 16)] = vals * ew16
        pltpu.sync_copy(msg_v.at[j], acc_sh.at[dst_v.at[j]], add=True)
        return carry

    lax.fori_loop(0, K, chunk, 0)

    plsc.subcore_barrier()

    @pl.when(s == 0)
    def _():
        pltpu.sync_copy(acc_sh, out_hbm.at[c])


_SC_PARAMS = pltpu.CompilerParams(
    needs_layout_passes=False, use_tc_tiling_on_sc=False)

_sc1 = functools.partial(
    pl.kernel,
    out_type=jax.ShapeDtypeStruct((NC, N), jnp.float32),
    mesh=plsc.VectorSubcoreMesh(core_axis_name="c", subcore_axis_name="s"),
    compiler_params=_SC_PARAMS,
    scratch_types=[
        pltpu.VMEM((N,), jnp.float32),
        pltpu.VMEM((K, C), jnp.int32),
        pltpu.VMEM((K, C), jnp.int32),
        pltpu.VMEM((K, C), jnp.float32),
        pltpu.VMEM((K, C), jnp.float32),
        pltpu.VMEM_SHARED((N,), jnp.float32),
        pltpu.SemaphoreType.DMA,
    ],
)(_sc1_body)


def _sc2_body(p_hbm, src_hbm, dst_hbm, ew_hbm, zeros_hbm, out_hbm,
              src_v, dst_v, ew_v, rows_v, acc_sh, sem):
    c = lax.axis_index("c")
    s = lax.axis_index("s")
    row = c * NS + s

    pltpu.sync_copy(src_hbm.at[row], src_v)
    pltpu.sync_copy(dst_hbm.at[row], dst_v)
    pltpu.sync_copy(ew_hbm.at[row], ew_v)

    @pl.when(s == 0)
    def _():
        pltpu.sync_copy(zeros_hbm, acc_sh)

    plsc.subcore_barrier()

    def chunk(j, carry):
        j_vec = jnp.broadcast_to(j, (16,)).astype(jnp.int32)
        pltpu.async_copy(p_hbm.at[src_v.at[j]], rows_v, sem).wait()
        for e in range(C):
            scale = plsc.load_gather(
                ew_v, [j_vec, jnp.full((16,), e, jnp.int32)])
            rows_v[e, :] = rows_v[e, :] * scale
        pltpu.sync_copy(rows_v, acc_sh.at[dst_v.at[j]], add=True)
        return carry

    lax.fori_loop(0, K, chunk, 0)

    plsc.subcore_barrier()

    @pl.when(s == 0)
    def _():
        pltpu.sync_copy(acc_sh, out_hbm.at[c])


_sc2 = functools.partial(
    pl.kernel,
    out_type=jax.ShapeDtypeStruct((NC, N, H2), jnp.float32),
    mesh=plsc.VectorSubcoreMesh(core_axis_name="c", subcore_axis_name="s"),
    compiler_params=_SC_PARAMS,
    scratch_types=[
        pltpu.VMEM((K, C), jnp.int32),
        pltpu.VMEM((K, C), jnp.int32),
        pltpu.VMEM((K, C), jnp.float32),
        pltpu.VMEM((C, H2), jnp.float32),
        pltpu.VMEM_SHARED((N, H2), jnp.float32),
        pltpu.SemaphoreType.DMA,
    ],
)(_sc2_body)


R1 = N


def _tc1_body(parts_ref, x_ref, uvwt_ref, w2rel_ref, w2root_ref, p_ref, r_ref):
    a1 = parts_ref[0:1, :] + parts_ref[1:2, :]
    ones = jnp.ones((1, R1), dtype=jnp.float32)
    at = jnp.concatenate([a1, x_ref[...], ones], axis=0)
    ht = jnp.maximum(
        jnp.dot(uvwt_ref[...], at, preferred_element_type=jnp.float32), 0.0
    )
    dn = (((0,), (0,)), ((), ()))
    p_ref[...] = lax.dot_general(ht, w2rel_ref[...], dn,
                                 preferred_element_type=jnp.float32)
    r_ref[...] = lax.dot_general(ht, w2root_ref[...], dn,
                                 preferred_element_type=jnp.float32)


def _tc1(parts, x_row, uvwt, w2rel, w2root):
    return pl.pallas_call(
        _tc1_body,
        grid=(1,),
        in_specs=[
            pl.BlockSpec((NC, R1), lambda i: (0, i)),
            pl.BlockSpec((1, R1), lambda i: (0, i)),
            pl.BlockSpec((H1, 3), lambda i: (0, 0)),
            pl.BlockSpec((H1, H2), lambda i: (0, 0)),
            pl.BlockSpec((H1, H2), lambda i: (0, 0)),
        ],
        out_specs=[
            pl.BlockSpec((R1, H2), lambda i: (i, 0)),
            pl.BlockSpec((R1, H2), lambda i: (i, 0)),
        ],
        out_shape=[
            jax.ShapeDtypeStruct((N, H2), jnp.float32),
            jax.ShapeDtypeStruct((N, H2), jnp.float32),
        ],
    )(parts, x_row, uvwt, w2rel, w2root)


NR = N // 8


def _tc2_body(parts_ref, r_ref, b2t_ref, wlk_ref, bl8_ref, out_ref):
    z = jnp.maximum(
        parts_ref[0] + parts_ref[1] + r_ref[...] + b2t_ref[...], 0.0)
    out_ref[...] = (
        jnp.dot(z, wlk_ref[...], preferred_element_type=jnp.float32)
        + bl8_ref[...])


def _tc2(parts2, r_flat, b2t, wlk, bl8):
    return pl.pallas_call(
        _tc2_body,
        grid=(1,),
        in_specs=[
            pl.BlockSpec((NC, NR, 128), lambda i: (0, 0, 0)),
            pl.BlockSpec((NR, 128), lambda i: (0, 0)),
            pl.BlockSpec((1, 128), lambda i: (0, 0)),
            pl.BlockSpec((128, 8), lambda i: (0, 0)),
            pl.BlockSpec((1, 8), lambda i: (0, 0)),
        ],
        out_specs=pl.BlockSpec((NR, 8), lambda i: (0, 0)),
        out_shape=jax.ShapeDtypeStruct((NR, 8), jnp.float32),
    )(parts2, r_flat, b2t, wlk, bl8)


def kernel(x, edge_index, edge_weight, W1_rel, b1_rel, W1_root,
           bn_gamma, bn_beta, bn_mean, bn_var,
           W2_rel, b2_rel, W2_root, Wl, bl):
    f32 = jnp.float32
    bn_scale = bn_gamma * lax.rsqrt(bn_var + BN_EPS)
    u = W1_rel[0] * bn_scale
    v = W1_root[0] * bn_scale
    w = (b1_rel - bn_mean) * bn_scale + bn_beta
    uvwt = jnp.stack([u, v, w], axis=1).astype(f32)

    src3 = edge_index[0].reshape(TILES, K, C)
    dst3 = edge_index[1].reshape(TILES, K, C)
    ew3 = edge_weight.reshape(TILES, K, C).astype(f32)

    x_flat = x.reshape(N).astype(f32)
    zeros_n = jnp.zeros((N,), f32)
    zeros_nh = jnp.zeros((N, H2), f32)

    parts1 = _sc1(x_flat, src3, dst3, ew3, zeros_n)
    p, r = _tc1(parts1, x_flat.reshape(1, N), uvwt, W2_rel, W2_root)
    parts2 = _sc2(p, src3, dst3, ew3, zeros_nh)

    b2t = jnp.tile(b2_rel, 8).reshape(1, 128)
    wlk = jnp.kron(jnp.eye(8, dtype=f32), Wl)
    bl8 = jnp.tile(bl, 8).reshape(1, 8)
    out = _tc2(parts2.reshape(NC, NR, 128), r.reshape(NR, 128),
               b2t, wlk, bl8)
    return out.reshape(N, 1)


# speedup vs baseline: 30.8869x; 1.4185x over previous
import functools

import jax
import jax.numpy as jnp
from jax import lax
from jax.experimental import pallas as pl
from jax.experimental.pallas import tpu as pltpu
from jax.experimental.pallas import tpu_sc as plsc

N = 10000
E = 320000
H1 = 256
H2 = 16
BN_EPS = 1e-5

NC = 2
NS = 16
TILES = NC * NS
EDGES_PER_TILE = E // TILES
C = 80
K = EDGES_PER_TILE // C


def _sc1_body(x_hbm, src_hbm, dst_hbm, ew_hbm, zeros_hbm, out_hbm,
              x_v, src_v, dst_v, ew_v, msg_v, acc_sh, sem):
    c = lax.axis_index("c")
    s = lax.axis_index("s")
    row = c * NS + s

    pltpu.sync_copy(x_hbm, x_v)
    pltpu.sync_copy(src_hbm.at[row], src_v)
    pltpu.sync_copy(dst_hbm.at[row], dst_v)
    pltpu.sync_copy(ew_hbm.at[row], ew_v)

    @pl.when(s == 0)
    def _():
        pltpu.sync_copy(zeros_hbm, acc_sh)

    plsc.subcore_barrier()

    def chunk(j, carry):
        for g in range(C // 16):
            idx = src_v[j, pl.ds(g * 16, 16)]
            ew16 = ew_v[j, pl.ds(g * 16, 16)]
            vals = plsc.load_gather(x_v, [idx])
            msg_v[j, pl.ds(g * 16, 16)] = vals * ew16
        pltpu.async_copy(msg_v.at[j], acc_sh.at[dst_v.at[j]], sem, add=True)
        return carry

    lax.fori_loop(0, K, chunk, 0)

    def drain(j, carry):
        pltpu.make_async_copy(msg_v.at[j], acc_sh.at[dst_v.at[j]], sem).wait()
        return carry

    lax.fori_loop(0, K, drain, 0)

    plsc.subcore_barrier()

    @pl.when(s == 0)
    def _():
        pltpu.sync_copy(acc_sh, out_hbm.at[c])


_SC_PARAMS = pltpu.CompilerParams(
    needs_layout_passes=False, use_tc_tiling_on_sc=False)

_sc1 = functools.partial(
    pl.kernel,
    out_type=jax.ShapeDtypeStruct((NC, N), jnp.float32),
    mesh=plsc.VectorSubcoreMesh(core_axis_name="c", subcore_axis_name="s"),
    compiler_params=_SC_PARAMS,
    scratch_types=[
        pltpu.VMEM((N,), jnp.float32),
        pltpu.VMEM((K, C), jnp.int32),
        pltpu.VMEM((K, C), jnp.int32),
        pltpu.VMEM((K, C), jnp.float32),
        pltpu.VMEM((K, C), jnp.float32),
        pltpu.VMEM_SHARED((N,), jnp.float32),
        pltpu.SemaphoreType.DMA,
    ],
)(_sc1_body)


def _sc2_body(p_hbm, src_hbm, dst_hbm, ew_hbm, zeros_hbm, out_hbm,
              src_v, dst_v, ew_v, rows0_v, rows1_v, acc_sh, gsem0, gsem1):
    c = lax.axis_index("c")
    s = lax.axis_index("s")
    row = c * NS + s

    pltpu.sync_copy(src_hbm.at[row], src_v)
    pltpu.sync_copy(dst_hbm.at[row], dst_v)
    pltpu.sync_copy(ew_hbm.at[row], ew_v)

    @pl.when(s == 0)
    def _():
        pltpu.sync_copy(zeros_hbm, acc_sh)

    plsc.subcore_barrier()

    def start_gather(j, rows_v, gsem):
        pltpu.async_copy(p_hbm.at[src_v.at[j]], rows_v, gsem)

    def wait_gather(j, rows_v, gsem):
        pltpu.make_async_copy(p_hbm.at[src_v.at[j]], rows_v, gsem).wait()

    def scale_and_scatter(j, rows_v):
        for g in range(C // 16):
            ew16 = ew_v[j, pl.ds(g * 16, 16)]
            for l in range(16):
                e = g * 16 + l
                scale = lax.gather(
                    ew16, jnp.full((16, 1), l, jnp.int32),
                    dimension_numbers=lax.GatherDimensionNumbers(
                        offset_dims=(), collapsed_slice_dims=(0,),
                        start_index_map=(0,)),
                    slice_sizes=(1,),
                    mode=lax.GatherScatterMode.PROMISE_IN_BOUNDS)
                rows_v[e, :] = rows_v[e, :] * scale
        pltpu.sync_copy(rows_v, acc_sh.at[dst_v.at[j]], add=True)

    start_gather(0, rows0_v, gsem0)

    def chunk2(j2, carry):
        j = 2 * j2
        wait_gather(j, rows0_v, gsem0)
        start_gather(j + 1, rows1_v, gsem1)
        scale_and_scatter(j, rows0_v)
        wait_gather(j + 1, rows1_v, gsem1)
        start_gather(j + 2, rows0_v, gsem0)
        scale_and_scatter(j + 1, rows1_v)
        return carry

    lax.fori_loop(0, (K - 1) // 2, chunk2, 0)
    wait_gather(K - 1, rows0_v, gsem0)
    scale_and_scatter(K - 1, rows0_v)

    plsc.subcore_barrier()

    @pl.when(s == 0)
    def _():
        pltpu.sync_copy(acc_sh, out_hbm.at[c])


_sc2 = functools.partial(
    pl.kernel,
    out_type=jax.ShapeDtypeStruct((NC, N, H2), jnp.float32),
    mesh=plsc.VectorSubcoreMesh(core_axis_name="c", subcore_axis_name="s"),
    compiler_params=_SC_PARAMS,
    scratch_types=[
        pltpu.VMEM((K, C), jnp.int32),
        pltpu.VMEM((K, C), jnp.int32),
        pltpu.VMEM((K, C), jnp.float32),
        pltpu.VMEM((C, H2), jnp.float32),
        pltpu.VMEM((C, H2), jnp.float32),
        pltpu.VMEM_SHARED((N, H2), jnp.float32),
        pltpu.SemaphoreType.DMA,
        pltpu.SemaphoreType.DMA,
    ],
)(_sc2_body)


R1 = N


def _tc1_body(parts_ref, x_ref, uvwt_ref, w2rel_ref, w2root_ref, p_ref, r_ref):
    a1 = parts_ref[0:1, :] + parts_ref[1:2, :]
    ones = jnp.ones((1, R1), dtype=jnp.float32)
    at = jnp.concatenate([a1, x_ref[...], ones], axis=0)
    ht = jnp.maximum(
        jnp.dot(uvwt_ref[...], at, preferred_element_type=jnp.float32), 0.0
    )
    dn = (((0,), (0,)), ((), ()))
    p_ref[...] = lax.dot_general(ht, w2rel_ref[...], dn,
                                 preferred_element_type=jnp.float32)
    r_ref[...] = lax.dot_general(ht, w2root_ref[...], dn,
                                 preferred_element_type=jnp.float32)


def _tc1(parts, x_row, uvwt, w2rel, w2root):
    return pl.pallas_call(
        _tc1_body,
        grid=(1,),
        in_specs=[
            pl.BlockSpec((NC, R1), lambda i: (0, i)),
            pl.BlockSpec((1, R1), lambda i: (0, i)),
            pl.BlockSpec((H1, 3), lambda i: (0, 0)),
            pl.BlockSpec((H1, H2), lambda i: (0, 0)),
            pl.BlockSpec((H1, H2), lambda i: (0, 0)),
        ],
        out_specs=[
            pl.BlockSpec((R1, H2), lambda i: (i, 0)),
            pl.BlockSpec((R1, H2), lambda i: (i, 0)),
        ],
        out_shape=[
            jax.ShapeDtypeStruct((N, H2), jnp.float32),
            jax.ShapeDtypeStruct((N, H2), jnp.float32),
        ],
    )(parts, x_row, uvwt, w2rel, w2root)


NR = N // 8


def _tc2_body(parts_ref, r_ref, b2t_ref, wlk_ref, bl8_ref, out_ref):
    z = jnp.maximum(
        parts_ref[0] + parts_ref[1] + r_ref[...] + b2t_ref[...], 0.0)
    out_ref[...] = (
        jnp.dot(z, wlk_ref[...], preferred_element_type=jnp.float32)
        + bl8_ref[...])


def _tc2(parts2, r_flat, b2t, wlk, bl8):
    return pl.pallas_call(
        _tc2_body,
        grid=(1,),
        in_specs=[
            pl.BlockSpec((NC, NR, 128), lambda i: (0, 0, 0)),
            pl.BlockSpec((NR, 128), lambda i: (0, 0)),
            pl.BlockSpec((1, 128), lambda i: (0, 0)),
            pl.BlockSpec((128, 8), lambda i: (0, 0)),
            pl.BlockSpec((1, 8), lambda i: (0, 0)),
        ],
        out_specs=pl.BlockSpec((NR, 8), lambda i: (0, 0)),
        out_shape=jax.ShapeDtypeStruct((NR, 8), jnp.float32),
    )(parts2, r_flat, b2t, wlk, bl8)


def kernel(x, edge_index, edge_weight, W1_rel, b1_rel, W1_root,
           bn_gamma, bn_beta, bn_mean, bn_var,
           W2_rel, b2_rel, W2_root, Wl, bl):
    f32 = jnp.float32
    bn_scale = bn_gamma * lax.rsqrt(bn_var + BN_EPS)
    u = W1_rel[0] * bn_scale
    v = W1_root[0] * bn_scale
    w = (b1_rel - bn_mean) * bn_scale + bn_beta
    uvwt = jnp.stack([u, v, w], axis=1).astype(f32)

    src3 = edge_index[0].reshape(TILES, K, C)
    dst3 = edge_index[1].reshape(TILES, K, C)
    ew3 = edge_weight.reshape(TILES, K, C).astype(f32)

    x_flat = x.reshape(N).astype(f32)
    zeros_n = jnp.zeros((N,), f32)
    zeros_nh = jnp.zeros((N, H2), f32)

    parts1 = _sc1(x_flat, src3, dst3, ew3, zeros_n)
    p, r = _tc1(parts1, x_flat.reshape(1, N), uvwt, W2_rel, W2_root)
    parts2 = _sc2(p, src3, dst3, ew3, zeros_nh)

    b2t = jnp.tile(b2_rel, 8).reshape(1, 128)
    wlk = jnp.kron(jnp.eye(8, dtype=f32), Wl)
    bl8 = jnp.tile(bl, 8).reshape(1, 8)
    out = _tc2(parts2.reshape(NC, NR, 128), r.reshape(NR, 128),
               b2t, wlk, bl8)
    return out.reshape(N, 1)
